# baseline (speedup 1.0000x reference)
import functools

import jax
import jax.numpy as jnp
from jax.experimental import pallas as pl
from jax.experimental.pallas import tpu as pltpu

BM1 = 96
BM2 = 128


def _fused_kernel(a_ref, x_ref, wcat_ref, bsum_ref, w2_ref, adj_ref, b2_ref,
                  o_ref, s_ref, g_ref, *, n_ma, n):
    i = pl.program_id(0)
    h = w2_ref.shape[0]

    @pl.when(i == 0)
    def _():
        x = x_ref[...]
        for k in range(3):
            s_ref[k] = jnp.dot(x, wcat_ref[:, k * h:(k + 1) * h],
                               preferred_element_type=jnp.float32)

    @pl.when(i < n_ma)
    def _():
        acc = bsum_ref[...]
        for k in range(3):
            acc = acc + jnp.dot(a_ref[k], s_ref[k],
                                preferred_element_type=jnp.float32)
        hid = jnp.maximum(acc, 0.0)
        g_ref[pl.ds(i * BM1, BM1), :] = jnp.dot(
            hid, w2_ref[...], preferred_element_type=jnp.float32)

    @pl.when(i >= n_ma)
    def _():
        z = jnp.dot(adj_ref[...], g_ref[:n, :],
                    preferred_element_type=jnp.float32)
        z = z + b2_ref[...]
        m = jnp.max(z, axis=1, keepdims=True)
        e = jnp.exp(z - m)
        lse = m + jnp.log(jnp.sum(e, axis=1, keepdims=True))
        o_ref[...] = z - lse


@jax.jit
def kernel(x, adj, sub_adj, W1_1, b1_1, W1_2, b1_2, W1_3, b1_3, W2, b2):
    n, f = x.shape
    h = W1_1.shape[1]
    c = W2.shape[1]
    n_ma = pl.cdiv(n, BM1)
    n_mb = pl.cdiv(n, BM2)

    wcat = jnp.concatenate([W1_1, W1_2, W1_3], axis=1)
    bsum = (b1_1 + b1_2 + b1_3).reshape(1, h)
    b2r = b2.reshape(1, c)

    out = pl.pallas_call(
        functools.partial(_fused_kernel, n_ma=n_ma, n=n),
        grid=(n_ma + n_mb,),
        in_specs=[
            pl.BlockSpec((3, BM1, n),
                         lambda i: (0, jnp.minimum(i, n_ma - 1), 0)),
            pl.BlockSpec((n, f), lambda i: (0, 0)),
            pl.BlockSpec((f, 3 * h), lambda i: (0, 0)),
            pl.BlockSpec((1, h), lambda i: (0, 0)),
            pl.BlockSpec((h, c), lambda i: (0, 0)),
            pl.BlockSpec((BM2, n), lambda i: (jnp.maximum(i - n_ma, 0), 0)),
            pl.BlockSpec((1, c), lambda i: (0, 0)),
        ],
        out_specs=pl.BlockSpec((BM2, c), lambda i: (jnp.maximum(i - n_ma, 0), 0)),
        out_shape=jax.ShapeDtypeStruct((n, c), jnp.float32),
        scratch_shapes=[
            pltpu.VMEM((3, n, h), jnp.float32),
            pltpu.VMEM((n_ma * BM1, c), jnp.float32),
        ],
        compiler_params=pltpu.CompilerParams(
            dimension_semantics=("arbitrary",)),
    )(sub_adj, x, wcat, bsum, W2, adj, b2r)

    return out
